# Initial kernel scaffold; baseline (speedup 1.0000x reference)
import jax
import jax.numpy as jnp
from jax.experimental import pallas as pl


def kernel(x, weight, bias):
    raise NotImplementedError("write your pallas kernel here")


# speedup vs baseline: 2.4592x; 2.4592x over previous
import functools

import jax
import jax.numpy as jnp
from jax.experimental import pallas as pl

NUM_EXPERTS = 64
TOP_K = 2
BT = 1024


def _gate_kernel(x_ref, w_ref, b_ref, weights_ref, indices_ref):
    x = x_ref[...]
    w = w_ref[...]
    b = b_ref[...]
    scores = jax.lax.dot_general(
        x, w, (((1,), (1,)), ((), ())), preferred_element_type=jnp.float32
    )
    m = jnp.max(scores, axis=-1, keepdims=True)
    e = jnp.exp(scores - m)
    p = e / jnp.sum(e, axis=-1, keepdims=True)
    biased = p + b[None, :]
    iota = jax.lax.broadcasted_iota(jnp.int32, biased.shape, 1)
    m1 = jnp.max(biased, axis=-1, keepdims=True)
    is1 = biased == m1
    i1 = jnp.min(jnp.where(is1, iota, NUM_EXPERTS), axis=-1, keepdims=True)
    sel1 = iota == i1
    masked = jnp.where(sel1, -jnp.inf, biased)
    m2 = jnp.max(masked, axis=-1, keepdims=True)
    is2 = masked == m2
    i2 = jnp.min(jnp.where(is2, iota, NUM_EXPERTS), axis=-1, keepdims=True)
    sel2 = iota == i2
    w1 = jnp.sum(jnp.where(sel1, p, 0.0), axis=-1, keepdims=True)
    w2 = jnp.sum(jnp.where(sel2, p, 0.0), axis=-1, keepdims=True)
    weights_ref[...] = jnp.concatenate([w1, w2], axis=1)
    indices_ref[...] = jnp.concatenate([i1, i2], axis=1)


@jax.jit
def kernel(x, weight, bias):
    tokens = x.shape[0]
    grid = (tokens // BT,)
    weights, indices = pl.pallas_call(
        _gate_kernel,
        grid=grid,
        in_specs=[
            pl.BlockSpec((BT, x.shape[1]), lambda i: (i, 0)),
            pl.BlockSpec(weight.shape, lambda i: (0, 0)),
            pl.BlockSpec(bias.shape, lambda i: (0,)),
        ],
        out_specs=[
            pl.BlockSpec((BT, TOP_K), lambda i: (i, 0)),
            pl.BlockSpec((BT, TOP_K), lambda i: (i, 0)),
        ],
        out_shape=[
            jax.ShapeDtypeStruct((tokens, TOP_K), jnp.float32),
            jax.ShapeDtypeStruct((tokens, TOP_K), jnp.int32),
        ],
    )(x, weight, bias)
    return weights, indices
